# baseline (speedup 1.0000x reference)
import jax
import jax.numpy as jnp
from jax.experimental import pallas as pl
from jax.experimental.pallas import tpu as pltpu

_S_BLK = 512


def _adapter_block(pids_ref, x_ref, wd_ref, bd_ref, wu_ref, bu_ref, g_ref,
                   b_ref, o_ref):
    wd = wd_ref[0].astype(jnp.bfloat16)
    wu = wu_ref[0].astype(jnp.bfloat16)
    half = _S_BLK // 4
    for i in range(4):
        x = x_ref[0, pl.ds(i * half, half), :]
        h = x.shape[-1]
        z = jnp.dot(x.astype(jnp.bfloat16), wd,
                    preferred_element_type=jnp.float32)
        z = jnp.maximum(z + bd_ref[0], 0.0)
        u = jnp.dot(z.astype(jnp.bfloat16), wu,
                    preferred_element_type=jnp.float32)
        y = x + u + bu_ref[0]
        mean = jnp.sum(y, axis=-1, keepdims=True) * (1.0 / h)
        msq = jnp.sum(y * y, axis=-1, keepdims=True) * (1.0 / h)
        var = msq - mean * mean
        scale = jax.lax.rsqrt(var + 1e-5)
        o_ref[0, pl.ds(i * half, half), :] = ((y - mean) * scale * g_ref[0]
                                              + b_ref[0])


def kernel(hidden, profile_ids, Wd, bd, Wu, bu, gamma, beta):
    B, S, H = hidden.shape
    P, _, K = Wd.shape
    pids = profile_ids.astype(jnp.int32)
    bd3 = bd.reshape(P, 1, K)
    bu3 = bu.reshape(P, 1, H)
    g3 = gamma.reshape(P, 1, H)
    b3 = beta.reshape(P, 1, H)

    grid = (B, S // _S_BLK)
    spec = pltpu.PrefetchScalarGridSpec(
        num_scalar_prefetch=1,
        grid=grid,
        in_specs=[
            pl.BlockSpec((1, _S_BLK, H), lambda b, s, pids: (b, s, 0)),
            pl.BlockSpec((1, H, K), lambda b, s, pids: (pids[b], 0, 0)),
            pl.BlockSpec((1, 1, K), lambda b, s, pids: (pids[b], 0, 0)),
            pl.BlockSpec((1, K, H), lambda b, s, pids: (pids[b], 0, 0)),
            pl.BlockSpec((1, 1, H), lambda b, s, pids: (pids[b], 0, 0)),
            pl.BlockSpec((1, 1, H), lambda b, s, pids: (pids[b], 0, 0)),
            pl.BlockSpec((1, 1, H), lambda b, s, pids: (pids[b], 0, 0)),
        ],
        out_specs=pl.BlockSpec((1, _S_BLK, H), lambda b, s, pids: (b, s, 0)),
    )
    return pl.pallas_call(
        _adapter_block,
        grid_spec=spec,
        out_shape=jax.ShapeDtypeStruct((B, S, H), jnp.float32),
        compiler_params=pltpu.CompilerParams(
            dimension_semantics=("parallel", "parallel")),
    )(pids, hidden, Wd, bd3, Wu, bu3, g3, b3)


# speedup vs baseline: 1.0625x; 1.0625x over previous
import jax
import jax.numpy as jnp
from jax.experimental import pallas as pl
from jax.experimental.pallas import tpu as pltpu

_S_BLK = 1024


def _adapter_block(pids_ref, x_ref, wd_ref, bd_ref, wu_ref, bu_ref, g_ref,
                   b_ref, o_ref):
    wd = wd_ref[0].astype(jnp.bfloat16)
    wu = wu_ref[0].astype(jnp.bfloat16)
    half = _S_BLK // 4
    for i in range(4):
        x = x_ref[0, pl.ds(i * half, half), :]
        h = x.shape[-1]
        z = jnp.dot(x.astype(jnp.bfloat16), wd,
                    preferred_element_type=jnp.float32)
        z = jnp.maximum(z + bd_ref[0], 0.0)
        u = jnp.dot(z.astype(jnp.bfloat16), wu,
                    preferred_element_type=jnp.float32)
        y = x + u + bu_ref[0]
        mean = jnp.sum(y, axis=-1, keepdims=True) * (1.0 / h)
        msq = jnp.sum(y * y, axis=-1, keepdims=True) * (1.0 / h)
        var = msq - mean * mean
        scale = jax.lax.rsqrt(var + 1e-5)
        o_ref[0, pl.ds(i * half, half), :] = ((y - mean) * scale * g_ref[0]
                                              + b_ref[0])


def kernel(hidden, profile_ids, Wd, bd, Wu, bu, gamma, beta):
    B, S, H = hidden.shape
    P, _, K = Wd.shape
    pids = profile_ids.astype(jnp.int32)
    bd3 = bd.reshape(P, 1, K)
    bu3 = bu.reshape(P, 1, H)
    g3 = gamma.reshape(P, 1, H)
    b3 = beta.reshape(P, 1, H)

    grid = (B, S // _S_BLK)
    spec = pltpu.PrefetchScalarGridSpec(
        num_scalar_prefetch=1,
        grid=grid,
        in_specs=[
            pl.BlockSpec((1, _S_BLK, H), lambda b, s, pids: (b, s, 0)),
            pl.BlockSpec((1, H, K), lambda b, s, pids: (pids[b], 0, 0)),
            pl.BlockSpec((1, 1, K), lambda b, s, pids: (pids[b], 0, 0)),
            pl.BlockSpec((1, K, H), lambda b, s, pids: (pids[b], 0, 0)),
            pl.BlockSpec((1, 1, H), lambda b, s, pids: (pids[b], 0, 0)),
            pl.BlockSpec((1, 1, H), lambda b, s, pids: (pids[b], 0, 0)),
            pl.BlockSpec((1, 1, H), lambda b, s, pids: (pids[b], 0, 0)),
        ],
        out_specs=pl.BlockSpec((1, _S_BLK, H), lambda b, s, pids: (b, s, 0)),
    )
    return pl.pallas_call(
        _adapter_block,
        grid_spec=spec,
        out_shape=jax.ShapeDtypeStruct((B, S, H), jnp.float32),
        compiler_params=pltpu.CompilerParams(
            dimension_semantics=("parallel", "parallel")),
    )(pids, hidden, Wd, bd3, Wu, bu3, g3, b3)
